# Initial kernel scaffold; baseline (speedup 1.0000x reference)
import jax
import jax.numpy as jnp
from jax.experimental import pallas as pl


def kernel(indices, table, W, b):
    raise NotImplementedError("write your pallas kernel here")


# speedup vs baseline: 1.4441x; 1.4441x over previous
import functools

import jax
import jax.numpy as jnp
from jax import lax
from jax.experimental import pallas as pl
from jax.experimental.pallas import tpu as pltpu
from jax.experimental.pallas import tpu_sc as plsc

_VOCAB = 100000
_EMBED = 128
_HIDDEN = 256
_NC = 2
_NS = 16
_NW = _NC * _NS
_CH = 128


def _proj_body(t_ref, w_ref, b_ref, o_ref):
    o_ref[...] = lax.dot_general(
        t_ref[...], w_ref[...], (((1,), (1,)), ((), ())),
        preferred_element_type=jnp.float32) + b_ref[...]


def _project_table(table, W, b):
    rows = 2000
    return pl.pallas_call(
        _proj_body,
        grid=(_VOCAB // rows,),
        in_specs=[
            pl.BlockSpec((rows, _EMBED), lambda i: (i, 0)),
            pl.BlockSpec((_HIDDEN, _EMBED), lambda i: (0, 0)),
            pl.BlockSpec((1, _HIDDEN), lambda i: (0, 0)),
        ],
        out_specs=pl.BlockSpec((rows, _HIDDEN), lambda i: (i, 0)),
        out_shape=jax.ShapeDtypeStruct((_VOCAB, _HIDDEN), jnp.float32),
    )(table, W, b.reshape(1, _HIDDEN))


@functools.lru_cache(maxsize=None)
def _make_gather(n_rows):
    bpw = n_rows // _NW
    nchunk = bpw // _CH
    mesh = plsc.VectorSubcoreMesh(core_axis_name="c", subcore_axis_name="s",
                                  num_cores=_NC, num_subcores=_NS)

    @functools.partial(
        pl.kernel,
        out_type=jax.ShapeDtypeStruct((n_rows, _HIDDEN), jnp.float32),
        mesh=mesh,
        scratch_types=[
            pltpu.VMEM((nchunk, _CH), jnp.int32),
            pltpu.VMEM((_CH, _HIDDEN), jnp.float32),
            pltpu.SemaphoreType.DMA,
        ],
    )
    def gather(idx_hbm, p_hbm, out_hbm, idx_v, rows_v, gsem):
        wid = lax.axis_index("s") * _NC + lax.axis_index("c")
        base = wid * bpw
        pltpu.sync_copy(idx_hbm.at[wid], idx_v)

        def body(c, carry):
            pltpu.async_copy(p_hbm.at[idx_v.at[c]], rows_v, gsem).wait()
            pltpu.sync_copy(rows_v, out_hbm.at[pl.ds(base + c * _CH, _CH)])
            return carry

        lax.fori_loop(0, nchunk, body, 0)

    return gather


def kernel(indices, table, W, b):
    bsz, hist = indices.shape
    n = bsz * hist
    proj = _project_table(table, W, b)
    idx = indices.astype(jnp.int32).reshape(_NW, n // _NW // _CH, _CH)
    out = _make_gather(n)(idx, proj)
    return out.reshape(bsz, hist, _HIDDEN)


# speedup vs baseline: 2.2833x; 1.5811x over previous
import functools

import jax
import jax.numpy as jnp
from jax import lax
from jax.experimental import pallas as pl
from jax.experimental.pallas import tpu as pltpu
from jax.experimental.pallas import tpu_sc as plsc

_VOCAB = 100000
_EMBED = 128
_HIDDEN = 256
_NC = 2
_NS = 16
_NW = _NC * _NS
_CH = 128


def _mm_body(e_ref, w_ref, b_ref, o_ref):
    res = lax.dot_general(
        e_ref[...], w_ref[...], (((1,), (1,)), ((), ())),
        preferred_element_type=jnp.float32) + b_ref[...]
    o_ref[...] = res.reshape(o_ref.shape)


def _project_rows(emb, W, b, bsz, hist):
    br = 64
    return pl.pallas_call(
        _mm_body,
        grid=(bsz // br,),
        in_specs=[
            pl.BlockSpec((br * hist, _EMBED), lambda i: (i, 0)),
            pl.BlockSpec((_HIDDEN, _EMBED), lambda i: (0, 0)),
            pl.BlockSpec((1, _HIDDEN), lambda i: (0, 0)),
        ],
        out_specs=pl.BlockSpec((br, hist, _HIDDEN), lambda i: (i, 0, 0)),
        out_shape=jax.ShapeDtypeStruct((bsz, hist, _HIDDEN), jnp.float32),
    )(emb, W, b.reshape(1, _HIDDEN))


@functools.lru_cache(maxsize=None)
def _make_gather(n_rows, width):
    bpw = n_rows // _NW
    nchunk = bpw // _CH
    mesh = plsc.VectorSubcoreMesh(core_axis_name="c", subcore_axis_name="s",
                                  num_cores=_NC, num_subcores=_NS)

    @functools.partial(
        pl.kernel,
        out_type=jax.ShapeDtypeStruct((n_rows, width), jnp.float32),
        mesh=mesh,
        scratch_types=[
            pltpu.VMEM((nchunk, _CH), jnp.int32),
            pltpu.VMEM((_CH, width), jnp.float32),
            pltpu.SemaphoreType.DMA,
        ],
    )
    def gather(idx_hbm, p_hbm, out_hbm, idx_v, rows_v, gsem):
        wid = lax.axis_index("s") * _NC + lax.axis_index("c")
        base = wid * bpw
        pltpu.sync_copy(idx_hbm.at[wid], idx_v)

        def body(c, carry):
            pltpu.async_copy(p_hbm.at[idx_v.at[c]], rows_v, gsem).wait()
            pltpu.sync_copy(rows_v, out_hbm.at[pl.ds(base + c * _CH, _CH)])
            return carry

        lax.fori_loop(0, nchunk, body, 0)

    return gather


def kernel(indices, table, W, b):
    bsz, hist = indices.shape
    n = bsz * hist
    idx = indices.astype(jnp.int32).reshape(_NW, n // _NW // _CH, _CH)
    emb = _make_gather(n, _EMBED)(idx, table)
    return _project_rows(emb, W, b, bsz, hist)
